# Initial kernel scaffold; baseline (speedup 1.0000x reference)
import jax
import jax.numpy as jnp
from jax.experimental import pallas as pl


def kernel(t, node_features, edge_index, edge_weight, batch_vector, W0, b0, g0, be0, rm0, rv0, W1, b1, g1, be1, rm1, rv1, Wf, bf):
    raise NotImplementedError("write your pallas kernel here")


# speedup vs baseline: 10.4148x; 10.4148x over previous
import functools

import jax
import jax.numpy as jnp
from jax import lax
from jax.experimental import pallas as pl
from jax.experimental.pallas import tpu as pltpu
from jax.experimental.pallas import tpu_sc as plsc

N = 10000
H = 128
EPS = 1e-5

NC = 2
NS = 16
L = 16
NW = NC * NS
CHUNK = 128
NP = 10240
RPT = NP // NS

_mesh = plsc.VectorSubcoreMesh(
    core_axis_name="c", subcore_axis_name="s", num_cores=NC, num_subcores=NS)
_sc_params = pltpu.CompilerParams(needs_layout_passes=False)


def _make_deg_kernel(nchunk):
  @functools.partial(
      pl.kernel,
      out_type=jax.ShapeDtypeStruct((NW * NP,), jnp.float32),
      mesh=_mesh,
      compiler_params=_sc_params,
      scratch_types=[
          pltpu.VMEM((nchunk, CHUNK), jnp.int32),
          pltpu.VMEM((nchunk, CHUNK), jnp.float32),
          pltpu.VMEM((NP,), jnp.float32),
      ],
  )
  def deg_kernel(col_hbm, ew_hbm, out_hbm, col_v, ew_v, deg_v):
    c = lax.axis_index("c")
    s = lax.axis_index("s")
    wid = c * NS + s
    pltpu.sync_copy(col_hbm.at[wid], col_v)
    pltpu.sync_copy(ew_hbm.at[wid], ew_v)

    zv = jnp.zeros((L,), jnp.float32)

    def zero_body(i, carry):
      deg_v[pl.ds(i * L, L)] = zv
      return carry

    lax.fori_loop(0, NP // L, zero_body, 0)

    def chunk_body(j, carry):
      for g in range(CHUNK // L):
        idx = col_v[j, pl.ds(g * L, L)]
        val = ew_v[j, pl.ds(g * L, L)]
        plsc.addupdate_scatter(deg_v, [idx], val)
      return carry

    lax.fori_loop(0, nchunk, chunk_body, 0)
    pltpu.sync_copy(deg_v, out_hbm.at[pl.ds(pl.multiple_of(wid * NP, 128), NP)])

  return deg_kernel


def _make_gather_scatter_kernel(nchunk):
  @functools.partial(
      pl.kernel,
      out_type=jax.ShapeDtypeStruct((NC, NP, H), jnp.float32),
      mesh=_mesh,
      compiler_params=_sc_params,
      scratch_types=[
          pltpu.VMEM((nchunk, CHUNK), jnp.int32),
          pltpu.VMEM((nchunk, CHUNK), jnp.int32),
          pltpu.VMEM((nchunk, CHUNK), jnp.float32),
          pltpu.VMEM((CHUNK, H), jnp.float32),
          pltpu.VMEM_SHARED((NP, H), jnp.float32),
          pltpu.SemaphoreType.DMA,
      ],
  )
  def gs_kernel(row_hbm, col_hbm, ew_hbm, y_hbm, out_hbm,
                row_v, col_v, ew_v, rows_v, acc_sh, sem):
    c = lax.axis_index("c")
    s = lax.axis_index("s")
    wid = c * NS + s
    pltpu.sync_copy(row_hbm.at[wid], row_v)
    pltpu.sync_copy(col_hbm.at[wid], col_v)
    pltpu.sync_copy(ew_hbm.at[wid], ew_v)

    zv = jnp.zeros((L,), jnp.float32)

    def zero_body(i, carry):
      for k in range(H // L):
        rows_v[i, pl.ds(k * L, L)] = zv
      return carry

    lax.fori_loop(0, CHUNK, zero_body, 0)
    base = pl.multiple_of(s * RPT, 128)
    for z in range(RPT // CHUNK):
      pltpu.sync_copy(rows_v, acc_sh.at[pl.ds(base + z * CHUNK, CHUNK)])
    plsc.subcore_barrier()

    def chunk_body(j, carry):
      pltpu.async_copy(y_hbm.at[row_v.at[j]], rows_v, sem).wait()

      def group_body(g, icarry):
        wv = ew_v[j, pl.ds(g * L, L)]
        base_i = g * L
        for i in range(L):
          wsplat = jnp.full((L,), wv[i], jnp.float32)
          for k in range(H // L):
            sl = pl.ds(k * L, L)
            rows_v[base_i + i, sl] = rows_v[base_i + i, sl] * wsplat
        return icarry

      lax.fori_loop(0, CHUNK // L, group_body, 0)
      pltpu.sync_copy(rows_v, acc_sh.at[col_v.at[j]], add=True)
      return carry

    lax.fori_loop(0, nchunk, chunk_body, 0)
    plsc.subcore_barrier()
    pltpu.sync_copy(acc_sh.at[pl.ds(base, RPT)], out_hbm.at[c, pl.ds(base, RPT)])

  return gs_kernel


def _stage0_body(deg_ref, x_ref, w_ref, y_ref, dis_ref):
  ones = jnp.ones((NW, 1), jnp.float32)
  degsum = lax.dot_general(deg_ref[...], ones, (((0,), (0,)), ((), ())),
                           preferred_element_type=jnp.float32)
  dis = lax.rsqrt(1.0 + degsum[:N])
  dis_ref[...] = dis
  xw = jnp.dot(x_ref[...], w_ref[...], preferred_element_type=jnp.float32)
  y_ref[...] = dis * xw


def _mid_body(s_ref, y_ref, dis_ref, b_ref, g_ref, be_ref, rm_ref, rv_ref,
              w_ref, y_next_ref):
  dis = dis_ref[...]
  sv = s_ref[...]
  conv = dis * (y_ref[...] + sv[0, :N] + sv[1, :N]) + b_ref[...]
  hsc = g_ref[...] * lax.rsqrt(rv_ref[...] + EPS)
  h = (conv - rm_ref[...]) * hsc + be_ref[...]
  h = jnp.maximum(h, 0.0)
  y_next_ref[...] = dis * jnp.dot(h, w_ref[...],
                                  preferred_element_type=jnp.float32)


def _final_body(s_ref, y_ref, dis_ref, b_ref, out_ref):
  sv = s_ref[...]
  out_ref[...] = dis_ref[...] * (y_ref[...] + sv[0, :N] + sv[1, :N]) + b_ref[...]


def kernel(t, node_features, edge_index, edge_weight, batch_vector,
           W0, b0, g0, be0, rm0, rv0,
           W1, b1, g1, be1, rm1, rv1,
           Wf, bf):
  row = edge_index[0].astype(jnp.int32)
  col = edge_index[1].astype(jnp.int32)
  e = row.shape[0]
  epw = -(-e // NW)
  epw = -(-epw // CHUNK) * CHUNK
  nchunk = epw // CHUNK
  pad = NW * epw - e
  row3 = jnp.pad(row, (0, pad)).reshape(NW, nchunk, CHUNK)
  col3 = jnp.pad(col, (0, pad)).reshape(NW, nchunk, CHUNK)
  ew3 = jnp.pad(edge_weight, (0, pad)).reshape(NW, nchunk, CHUNK)

  deg_parts = _make_deg_kernel(nchunk)(col3, ew3).reshape(NW, NP)

  y0, dis = pl.pallas_call(
      _stage0_body,
      out_shape=(jax.ShapeDtypeStruct((N, H), jnp.float32),
                 jax.ShapeDtypeStruct((N, 1), jnp.float32)),
  )(deg_parts, node_features, W0)

  gs = _make_gather_scatter_kernel(nchunk)

  s0 = gs(row3, col3, ew3, y0)
  y1 = pl.pallas_call(
      _mid_body,
      out_shape=jax.ShapeDtypeStruct((N, H), jnp.float32),
  )(s0, y0, dis, b0, g0, be0, rm0, rv0, W1)

  s1 = gs(row3, col3, ew3, y1)
  y2 = pl.pallas_call(
      _mid_body,
      out_shape=jax.ShapeDtypeStruct((N, H), jnp.float32),
  )(s1, y1, dis, b1, g1, be1, rm1, rv1, Wf)

  s2 = gs(row3, col3, ew3, y2)
  dz_dt = pl.pallas_call(
      _final_body,
      out_shape=jax.ShapeDtypeStruct((N, H), jnp.float32),
  )(s2, y2, dis, bf)

  return (dz_dt, edge_index, edge_weight, batch_vector)
